# baseline (speedup 1.0000x reference)
import functools
import jax
import jax.numpy as jnp
from jax.experimental import pallas as pl
from jax.experimental.pallas import tpu as pltpu

B, L = 8, 448
SEQ_EMB = 32
RES_DIM = 128
PAIR_DIM = 64
MAX_REL = 32
NREL = 2 * MAX_REL + 1
NUM_EMB = 5
IB = 16

_HI = jax.lax.Precision.HIGHEST


def _prep_body(seq_ref, dih_ref, ent_ref, acc_ref, con_ref, emb_ref, pe_ref,
               rel_emb_ref, Wr_ref, br_ref, Wp_ref, bp_ref,
               res_out, relproj_out):
    seq = seq_ref[...]
    onehot = (seq[..., None] ==
              jax.lax.broadcasted_iota(jnp.int32, (B, L, NUM_EMB), 2)
              ).astype(jnp.float32)
    M = jax.lax.dot_general(emb_ref[...], Wr_ref[0:SEQ_EMB, :],
                            (((1,), (0,)), ((), ())), precision=_HI)
    res = jax.lax.dot_general(onehot.reshape(B * L, NUM_EMB), M,
                              (((1,), (0,)), ((), ())), precision=_HI)
    res = res + jax.lax.dot_general(
        dih_ref[...].reshape(B * L, 4), Wr_ref[SEQ_EMB:SEQ_EMB + 4, :],
        (((1,), (0,)), ((), ())), precision=_HI)
    res = res.reshape(B, L, RES_DIM)
    res = res + ent_ref[...][..., None] * Wr_ref[SEQ_EMB + 4, :][None, None, :]
    res = res + acc_ref[...][..., None] * Wr_ref[SEQ_EMB + 5, :][None, None, :]
    res = res + con_ref[...][..., None] * Wr_ref[SEQ_EMB + 6, :][None, None, :]
    res = res + br_ref[...][None, None, :]
    res = res + pe_ref[0, :L, :][None]
    res_out[...] = res
    relproj_out[...] = jax.lax.dot_general(
        rel_emb_ref[...], Wp_ref[2:, :], (((1,), (0,)), ((), ())),
        precision=_HI) + bp_ref[...][None, :]


def _pair_body(relproj_ref, Wp_ref, p_ref, c_ref, out_ref, rel_scratch):
    i_blk = pl.program_id(0)
    b = pl.program_id(1)

    @pl.when(b == 0)
    def _():
        i0 = i_blk * IB
        j = jax.lax.broadcasted_iota(jnp.int32, (IB, L), 1)
        ii = jax.lax.broadcasted_iota(jnp.int32, (IB, L), 0) + i0
        idx = jnp.clip(j - ii + MAX_REL, 0, 2 * MAX_REL)
        onehot = (idx[..., None] ==
                  jax.lax.broadcasted_iota(jnp.int32, (IB, L, NREL), 2)
                  ).astype(jnp.float32)
        rel = jax.lax.dot_general(onehot.reshape(IB * L, NREL),
                                  relproj_ref[...],
                                  (((1,), (0,)), ((), ())), precision=_HI)
        rel_scratch[...] = rel.reshape(IB, L, PAIR_DIM)

    w0 = Wp_ref[0, :][None, None, :]
    w1 = Wp_ref[1, :][None, None, :]
    p = p_ref[0][..., None]
    c = c_ref[0][..., None]
    out_ref[0] = p * w0 + c * w1 + rel_scratch[...]


@jax.jit
def _impl(sequence_int, dihedral_features, pairing_probs, positional_entropy,
          coupling_matrix, accessibility, conservation, emb_table, pe,
          rel_emb, W_res, b_res, W_pair, b_pair):
    res, relproj = pl.pallas_call(
        _prep_body,
        out_shape=(
            jax.ShapeDtypeStruct((B, L, RES_DIM), jnp.float32),
            jax.ShapeDtypeStruct((NREL, PAIR_DIM), jnp.float32),
        ),
    )(sequence_int.astype(jnp.int32), dihedral_features, positional_entropy,
      accessibility, conservation, emb_table, pe, rel_emb, W_res, b_res,
      W_pair, b_pair)

    grid = (L // IB, B)
    pair = pl.pallas_call(
        _pair_body,
        grid=grid,
        in_specs=[
            pl.BlockSpec((NREL, PAIR_DIM), lambda i, b: (0, 0)),
            pl.BlockSpec((2, PAIR_DIM), lambda i, b: (0, 0)),
            pl.BlockSpec((1, IB, L), lambda i, b: (b, i, 0)),
            pl.BlockSpec((1, IB, L), lambda i, b: (b, i, 0)),
        ],
        out_specs=pl.BlockSpec((1, IB, L, PAIR_DIM),
                               lambda i, b: (b, i, 0, 0)),
        out_shape=jax.ShapeDtypeStruct((B, L, L, PAIR_DIM), jnp.float32),
        scratch_shapes=[pltpu.VMEM((IB, L, PAIR_DIM), jnp.float32)],
        compiler_params=pltpu.CompilerParams(
            dimension_semantics=("arbitrary", "arbitrary")),
    )(relproj, W_pair[0:2, :], pairing_probs, coupling_matrix)
    return res, pair


def kernel(sequence_int, mask, dihedral_features, pairing_probs,
           positional_entropy, coupling_matrix, accessibility, conservation,
           emb_table, pe, rel_emb, W_res, b_res, W_pair, b_pair):
    res, pair = _impl(sequence_int, dihedral_features, pairing_probs,
                      positional_entropy, coupling_matrix, accessibility,
                      conservation, emb_table, pe, rel_emb, W_res, b_res,
                      W_pair, b_pair)
    return res, pair, mask
